# Initial kernel scaffold; baseline (speedup 1.0000x reference)
import jax
import jax.numpy as jnp
from jax.experimental import pallas as pl


def kernel(x, Wr, W1, W2):
    raise NotImplementedError("write your pallas kernel here")


# speedup vs baseline: 1.3486x; 1.3486x over previous
import functools

import jax
import jax.numpy as jnp
from jax import lax
from jax.experimental import pallas as pl
from jax.experimental.pallas import tpu as pltpu
from jax.experimental.pallas import tpu_sc as plsc

D_MODEL = 1024
D_FF = 2048
N_EXPERTS = 8
CAP = 640
N_TOK = 2048
N_SLOT = N_EXPERTS * CAP
DUMP = N_SLOT
BUF_ROWS = N_SLOT + 8
NW = 32
TPW = N_TOK // NW
EPAD = 128
FB = 512
NFB = D_FF // FB


def _cumsum_tokens(oh):
    nchunk = N_TOK // 128
    ii = lax.broadcasted_iota(jnp.int32, (128, 128), 0)
    jj = lax.broadcasted_iota(jnp.int32, (128, 128), 1)
    tril = (jj <= ii).astype(jnp.float32)
    chunks, totals = [], []
    for c in range(nchunk):
        blk = oh[c * 128:(c + 1) * 128, :]
        chunks.append(lax.dot_general(tril, blk, (((1,), (0,)), ((), ()))))
        totals.append(jnp.sum(blk, axis=0, keepdims=True))
    tot = jnp.concatenate(totals, axis=0)
    i2 = lax.broadcasted_iota(jnp.int32, (nchunk, nchunk), 0)
    j2 = lax.broadcasted_iota(jnp.int32, (nchunk, nchunk), 1)
    stril = (j2 < i2).astype(jnp.float32)
    off = lax.dot_general(stril, tot, (((1,), (0,)), ((), ())))
    return jnp.concatenate(
        [chunks[c] + off[c:c + 1, :] for c in range(nchunk)], axis=0)


def _router_body(x_ref, wr_ref, gd0_ref, gd1_ref, gc0_ref, gc1_ref,
                 w0_ref, w1_ref, loss_ref):
    x = x_ref[...].astype(jnp.bfloat16)
    wr = wr_ref[...].astype(jnp.bfloat16)
    logits = lax.dot_general(x, wr, (((1,), (1,)), ((), ())),
                             preferred_element_type=jnp.float32)
    lane = lax.broadcasted_iota(jnp.int32, (N_TOK, EPAD), 1)
    neg = jnp.float32(-jnp.inf)
    logits = jnp.where(lane < N_EXPERTS, logits, neg)
    m = jnp.max(logits, axis=1, keepdims=True)
    p = jnp.exp(logits - m)
    gate = p / jnp.sum(p, axis=1, keepdims=True)

    v0 = jnp.max(gate, axis=1, keepdims=True)
    idx0 = jnp.min(jnp.where(gate == v0, lane, EPAD), axis=1, keepdims=True)
    m0 = lane == idx0
    gate1 = jnp.where(m0, -1.0, gate)
    v1 = jnp.max(gate1, axis=1, keepdims=True)
    idx1 = jnp.min(jnp.where(gate1 == v1, lane, EPAD), axis=1, keepdims=True)
    m1 = lane == idx1

    oh0 = m0.astype(jnp.float32)
    oh1 = m1.astype(jnp.float32)
    cumcat = _cumsum_tokens(jnp.concatenate([oh0, oh1], axis=1))
    cum0 = cumcat[:, :EPAD]
    cum1 = cumcat[:, EPAD:]
    pos0 = jnp.sum(cum0 * oh0, axis=1, keepdims=True)
    count0 = jnp.sum(oh0, axis=0, keepdims=True)
    u1 = jnp.minimum(count0, float(CAP))
    keep0 = pos0 <= float(CAP)
    g0 = idx0 * CAP + (pos0 - 1.0).astype(jnp.int32)
    gd0_ref[...] = jnp.where(keep0, g0, DUMP)
    gc0_ref[...] = jnp.where(keep0, g0, 0)
    w0_ref[...] = jnp.where(keep0, v0, 0.0)

    pos1 = jnp.sum(cum1 * oh1, axis=1, keepdims=True)
    start1 = jnp.sum(oh1 * u1, axis=1, keepdims=True)
    posp = pos1 + start1
    keep1 = posp <= float(CAP)
    g1 = idx1 * CAP + (posp - 1.0).astype(jnp.int32)
    gd1_ref[...] = jnp.where(keep1, g1, DUMP)
    gc1_ref[...] = jnp.where(keep1, g1, 0)
    w1_ref[...] = jnp.where(keep1, v1, 0.0)

    count1 = jnp.sum(oh1, axis=0, keepdims=True)
    u2 = u1 + jnp.minimum(count1, float(CAP) - u1)
    probe = jnp.mean(gate, axis=0, keepdims=True)
    frac = jnp.maximum(u2, 1e-9) / (N_TOK * 2 + 1e-9)
    loss_ref[...] = jnp.sum(probe * frac, axis=1, keepdims=True) * N_EXPERTS


def _run_router(flat_x, wr_pad):
    i32 = jnp.int32
    f32 = jnp.float32
    outs = pl.pallas_call(
        _router_body,
        out_shape=(
            jax.ShapeDtypeStruct((N_TOK, 1), i32),
            jax.ShapeDtypeStruct((N_TOK, 1), i32),
            jax.ShapeDtypeStruct((N_TOK, 1), i32),
            jax.ShapeDtypeStruct((N_TOK, 1), i32),
            jax.ShapeDtypeStruct((N_TOK, 1), f32),
            jax.ShapeDtypeStruct((N_TOK, 1), f32),
            jax.ShapeDtypeStruct((1, 1), f32),
        ),
    )(flat_x, wr_pad)
    return outs


def _make_dispatch():
    mesh = plsc.VectorSubcoreMesh(core_axis_name="c", subcore_axis_name="s")

    @functools.partial(
        pl.kernel,
        out_type=jax.ShapeDtypeStruct((BUF_ROWS, D_MODEL), jnp.float32),
        mesh=mesh,
        scratch_types=[
            pltpu.VMEM((TPW,), jnp.int32),
            pltpu.VMEM((TPW, D_MODEL), jnp.float32),
            pltpu.SemaphoreType.DMA,
        ],
    )
    def dispatch(x_hbm, g0_hbm, g1_hbm, buf_hbm, idx_v, rows_v, sem):
        wid = lax.axis_index("s") * 2 + lax.axis_index("c")
        pltpu.sync_copy(x_hbm.at[wid], rows_v)
        pltpu.sync_copy(g0_hbm.at[wid], idx_v)
        pltpu.async_copy(rows_v, buf_hbm.at[idx_v], sem).wait()
        pltpu.sync_copy(g1_hbm.at[wid], idx_v)
        pltpu.async_copy(rows_v, buf_hbm.at[idx_v], sem).wait()

    return dispatch


def _ffn_body(buf_ref, w1a_ref, w1b_ref, w2_ref, out_ref):
    fb = pl.program_id(1)
    xb = buf_ref[...].astype(jnp.bfloat16)
    w1a = w1a_ref[0]
    w1b = w1b_ref[0]
    w2 = w2_ref[0]
    cd = (((1,), (1,)), ((), ()))
    a = lax.dot_general(xb, w1a, cd, preferred_element_type=jnp.float32)
    b = lax.dot_general(xb, w1b, cd, preferred_element_type=jnp.float32)
    h = (a * jax.nn.sigmoid(a) * b).astype(jnp.bfloat16)
    part = lax.dot_general(h, w2, cd, preferred_element_type=jnp.float32)

    @pl.when(fb == 0)
    def _():
        out_ref[...] = part

    @pl.when(fb != 0)
    def _():
        out_ref[...] += part


def _run_ffn(buf, w1a, w1b, w2):
    return pl.pallas_call(
        _ffn_body,
        grid=(N_EXPERTS, NFB),
        in_specs=[
            pl.BlockSpec((CAP, D_MODEL), lambda e, fb: (e, 0)),
            pl.BlockSpec((1, FB, D_MODEL), lambda e, fb: (e, fb, 0)),
            pl.BlockSpec((1, FB, D_MODEL), lambda e, fb: (e, fb, 0)),
            pl.BlockSpec((1, D_MODEL, FB), lambda e, fb: (e, 0, fb)),
        ],
        compiler_params=pltpu.CompilerParams(
            dimension_semantics=("arbitrary", "arbitrary")),
        out_specs=pl.BlockSpec((CAP, D_MODEL), lambda e, fb: (e, 0)),
        out_shape=jax.ShapeDtypeStruct((N_SLOT, D_MODEL), jnp.float32),
    )(buf, w1a, w1b, w2)


HALF = TPW // 2


def _make_combine():
    mesh = plsc.VectorSubcoreMesh(core_axis_name="c", subcore_axis_name="s")

    @functools.partial(
        pl.kernel,
        out_type=jax.ShapeDtypeStruct((NW, TPW, D_MODEL), jnp.float32),
        mesh=mesh,
        scratch_types=[
            pltpu.VMEM((HALF,), jnp.int32),
            pltpu.VMEM((HALF, D_MODEL), jnp.float32),
            pltpu.VMEM((HALF, D_MODEL), jnp.float32),
            pltpu.VMEM((TPW, 16), jnp.float32),
            pltpu.VMEM((TPW, 16), jnp.float32),
            pltpu.SemaphoreType.DMA,
        ],
    )
    def combine(ob_hbm, g0_hbm, g1_hbm, w0_hbm, w1_hbm, y_hbm,
                idx_v, r0_v, r1_v, w0_v, w1_v, sem):
        wid = lax.axis_index("s") * 2 + lax.axis_index("c")
        pltpu.sync_copy(w0_hbm.at[wid], w0_v)
        pltpu.sync_copy(w1_hbm.at[wid], w1_v)
        for half in range(2):
            pltpu.sync_copy(g0_hbm.at[wid, pl.ds(half * HALF, HALF)], idx_v)
            pltpu.async_copy(ob_hbm.at[idx_v], r0_v, sem).wait()
            pltpu.sync_copy(g1_hbm.at[wid, pl.ds(half * HALF, HALF)], idx_v)
            pltpu.async_copy(ob_hbm.at[idx_v], r1_v, sem).wait()

            def row_body(i, _):
                wv0 = w0_v[half * HALF + i, :]
                wv1 = w1_v[half * HALF + i, :]
                k0 = wv0 != 0.0
                k1 = wv1 != 0.0

                def col_body(c, _):
                    a = r0_v[i, pl.ds(c * 16, 16)]
                    b = r1_v[i, pl.ds(c * 16, 16)]
                    res = (jnp.where(k0, a * wv0, 0.0)
                           + jnp.where(k1, b * wv1, 0.0))
                    r0_v[i, pl.ds(c * 16, 16)] = res
                    return 0

                lax.fori_loop(0, D_MODEL // 16, col_body, 0, unroll=4)
                return 0

            lax.fori_loop(0, HALF, row_body, 0)
            pltpu.sync_copy(r0_v, y_hbm.at[wid, pl.ds(half * HALF, HALF)])

    return combine


def kernel(x, Wr, W1, W2):
    B, T, H = x.shape
    flat_x = x.reshape(N_TOK, H)
    wr_pad = jnp.zeros((EPAD, H), jnp.float32).at[:N_EXPERTS].set(Wr)
    gd0, gd1, gc0, gc1, w0, w1, loss = _run_router(flat_x, wr_pad)

    x32 = flat_x.reshape(NW, TPW, H)
    gd0r = gd0.reshape(NW, TPW)
    gd1r = gd1.reshape(NW, TPW)
    buf = _make_dispatch()(x32, gd0r, gd1r)

    w1a = W1[:, :D_FF, :].astype(jnp.bfloat16)
    w1b = W1[:, D_FF:, :].astype(jnp.bfloat16)
    out_buf = _run_ffn(buf, w1a, w1b, W2.astype(jnp.bfloat16))

    gc0r = gc0.reshape(NW, TPW)
    gc1r = gc1.reshape(NW, TPW)
    w0s = jnp.broadcast_to(w0.reshape(N_TOK, 1), (N_TOK, 16)).reshape(NW, TPW, 16)
    w1s = jnp.broadcast_to(w1.reshape(N_TOK, 1), (N_TOK, 16)).reshape(NW, TPW, 16)
    y = _make_combine()(out_buf, gc0r, gc1r, w0s, w1s)

    return y.reshape(B, T, H), loss.reshape(())


# speedup vs baseline: 2.4706x; 1.8319x over previous
import functools

import jax
import jax.numpy as jnp
from jax import lax
from jax.experimental import pallas as pl
from jax.experimental.pallas import tpu as pltpu
from jax.experimental.pallas import tpu_sc as plsc

D_MODEL = 1024
D_FF = 2048
N_EXPERTS = 8
CAP = 640
N_TOK = 2048
N_SLOT = N_EXPERTS * CAP
DUMP = N_SLOT
BUF_ROWS = N_SLOT + 8
NW = 32
TPW = N_TOK // NW
EPAD = 128
FB = 1024
NFB = D_FF // FB


def _cumsum_tokens(oh):
    nchunk = N_TOK // 128
    ii = lax.broadcasted_iota(jnp.int32, (128, 128), 0)
    jj = lax.broadcasted_iota(jnp.int32, (128, 128), 1)
    tril = (jj <= ii).astype(jnp.float32)
    chunks, totals = [], []
    for c in range(nchunk):
        blk = oh[c * 128:(c + 1) * 128, :]
        chunks.append(lax.dot_general(tril, blk, (((1,), (0,)), ((), ()))))
        totals.append(jnp.sum(blk, axis=0, keepdims=True))
    tot = jnp.concatenate(totals, axis=0)
    i2 = lax.broadcasted_iota(jnp.int32, (nchunk, nchunk), 0)
    j2 = lax.broadcasted_iota(jnp.int32, (nchunk, nchunk), 1)
    stril = (j2 < i2).astype(jnp.float32)
    off = lax.dot_general(stril, tot, (((1,), (0,)), ((), ())))
    return jnp.concatenate(
        [chunks[c] + off[c:c + 1, :] for c in range(nchunk)], axis=0)


def _router_body(x_ref, wr_ref, gd0_ref, gd1_ref, gc0_ref, gc1_ref,
                 w0_ref, w1_ref, loss_ref):
    x = x_ref[...].astype(jnp.bfloat16)
    wr = wr_ref[...].astype(jnp.bfloat16)
    logits = lax.dot_general(x, wr, (((1,), (1,)), ((), ())),
                             preferred_element_type=jnp.float32)
    lane = lax.broadcasted_iota(jnp.int32, (N_TOK, EPAD), 1)
    neg = jnp.float32(-jnp.inf)
    logits = jnp.where(lane < N_EXPERTS, logits, neg)
    m = jnp.max(logits, axis=1, keepdims=True)
    p = jnp.exp(logits - m)
    gate = p / jnp.sum(p, axis=1, keepdims=True)

    v0 = jnp.max(gate, axis=1, keepdims=True)
    idx0 = jnp.min(jnp.where(gate == v0, lane, EPAD), axis=1, keepdims=True)
    m0 = lane == idx0
    gate1 = jnp.where(m0, -1.0, gate)
    v1 = jnp.max(gate1, axis=1, keepdims=True)
    idx1 = jnp.min(jnp.where(gate1 == v1, lane, EPAD), axis=1, keepdims=True)
    m1 = lane == idx1

    oh0 = m0.astype(jnp.float32)
    oh1 = m1.astype(jnp.float32)
    cumcat = _cumsum_tokens(jnp.concatenate([oh0, oh1], axis=1))
    cum0 = cumcat[:, :EPAD]
    cum1 = cumcat[:, EPAD:]
    pos0 = jnp.sum(cum0 * oh0, axis=1, keepdims=True)
    count0 = jnp.sum(oh0, axis=0, keepdims=True)
    u1 = jnp.minimum(count0, float(CAP))
    keep0 = pos0 <= float(CAP)
    g0 = idx0 * CAP + (pos0 - 1.0).astype(jnp.int32)
    gd0_ref[...] = jnp.where(keep0, g0, DUMP)
    gc0_ref[...] = jnp.where(keep0, g0, 0)
    w0_ref[...] = jnp.where(keep0, v0, 0.0)

    pos1 = jnp.sum(cum1 * oh1, axis=1, keepdims=True)
    start1 = jnp.sum(oh1 * u1, axis=1, keepdims=True)
    posp = pos1 + start1
    keep1 = posp <= float(CAP)
    g1 = idx1 * CAP + (posp - 1.0).astype(jnp.int32)
    gd1_ref[...] = jnp.where(keep1, g1, DUMP)
    gc1_ref[...] = jnp.where(keep1, g1, 0)
    w1_ref[...] = jnp.where(keep1, v1, 0.0)

    count1 = jnp.sum(oh1, axis=0, keepdims=True)
    u2 = u1 + jnp.minimum(count1, float(CAP) - u1)
    probe = jnp.mean(gate, axis=0, keepdims=True)
    frac = jnp.maximum(u2, 1e-9) / (N_TOK * 2 + 1e-9)
    loss_ref[...] = jnp.sum(probe * frac, axis=1, keepdims=True) * N_EXPERTS


def _run_router(flat_x, wr_pad):
    i32 = jnp.int32
    f32 = jnp.float32
    outs = pl.pallas_call(
        _router_body,
        out_shape=(
            jax.ShapeDtypeStruct((N_TOK, 1), i32),
            jax.ShapeDtypeStruct((N_TOK, 1), i32),
            jax.ShapeDtypeStruct((N_TOK, 1), i32),
            jax.ShapeDtypeStruct((N_TOK, 1), i32),
            jax.ShapeDtypeStruct((N_TOK, 1), f32),
            jax.ShapeDtypeStruct((N_TOK, 1), f32),
            jax.ShapeDtypeStruct((1, 1), f32),
        ),
    )(flat_x, wr_pad)
    return outs


def _make_dispatch():
    mesh = plsc.VectorSubcoreMesh(core_axis_name="c", subcore_axis_name="s")

    @functools.partial(
        pl.kernel,
        out_type=jax.ShapeDtypeStruct((BUF_ROWS, D_MODEL), jnp.float32),
        mesh=mesh,
        scratch_types=[
            pltpu.VMEM((TPW,), jnp.int32),
            pltpu.VMEM((TPW, D_MODEL), jnp.float32),
            pltpu.SemaphoreType.DMA,
        ],
    )
    def dispatch(x_hbm, g0_hbm, g1_hbm, buf_hbm, idx_v, rows_v, sem):
        wid = lax.axis_index("s") * 2 + lax.axis_index("c")
        pltpu.sync_copy(x_hbm.at[wid], rows_v)
        pltpu.sync_copy(g0_hbm.at[wid], idx_v)
        pltpu.async_copy(rows_v, buf_hbm.at[idx_v], sem).wait()
        pltpu.sync_copy(g1_hbm.at[wid], idx_v)
        pltpu.async_copy(rows_v, buf_hbm.at[idx_v], sem).wait()

    return dispatch


def _ffn_body(buf_ref, w1a_ref, w1b_ref, w2_ref, out_ref):
    fb = pl.program_id(1)
    xb = buf_ref[...].astype(jnp.bfloat16)
    w1a = w1a_ref[0].astype(jnp.bfloat16)
    w1b = w1b_ref[0].astype(jnp.bfloat16)
    w2 = w2_ref[0].astype(jnp.bfloat16)
    cd = (((1,), (1,)), ((), ()))
    a = lax.dot_general(xb, w1a, cd, preferred_element_type=jnp.float32)
    b = lax.dot_general(xb, w1b, cd, preferred_element_type=jnp.float32)
    h = (a * jax.nn.sigmoid(a) * b).astype(jnp.bfloat16)
    part = lax.dot_general(h, w2, cd, preferred_element_type=jnp.float32)

    @pl.when(fb == 0)
    def _():
        out_ref[...] = part

    @pl.when(fb != 0)
    def _():
        out_ref[...] += part


def _run_ffn(buf, w1a, w1b, w2):
    return pl.pallas_call(
        _ffn_body,
        grid=(N_EXPERTS, NFB),
        in_specs=[
            pl.BlockSpec((CAP, D_MODEL), lambda e, fb: (e, 0)),
            pl.BlockSpec((1, FB, D_MODEL), lambda e, fb: (e, fb, 0)),
            pl.BlockSpec((1, FB, D_MODEL), lambda e, fb: (e, fb + NFB, 0)),
            pl.BlockSpec((1, D_MODEL, FB), lambda e, fb: (e, 0, fb)),
        ],
        compiler_params=pltpu.CompilerParams(
            dimension_semantics=("arbitrary", "arbitrary")),
        out_specs=pl.BlockSpec((CAP, D_MODEL), lambda e, fb: (e, 0)),
        out_shape=jax.ShapeDtypeStruct((N_SLOT, D_MODEL), jnp.float32),
    )(buf, w1a, w1b, w2)


HALF = TPW // 2


def _make_combine():
    mesh = plsc.VectorSubcoreMesh(core_axis_name="c", subcore_axis_name="s")

    @functools.partial(
        pl.kernel,
        out_type=jax.ShapeDtypeStruct((NW, TPW, D_MODEL), jnp.float32),
        mesh=mesh,
        scratch_types=[
            pltpu.VMEM((HALF,), jnp.int32),
            pltpu.VMEM((HALF, D_MODEL), jnp.float32),
            pltpu.VMEM((HALF, D_MODEL), jnp.float32),
            pltpu.VMEM((TPW, 16), jnp.float32),
            pltpu.VMEM((TPW, 16), jnp.float32),
            pltpu.SemaphoreType.DMA,
        ],
    )
    def combine(ob_hbm, g0_hbm, g1_hbm, w0_hbm, w1_hbm, y_hbm,
                idx_v, r0_v, r1_v, w0_v, w1_v, sem):
        wid = lax.axis_index("s") * 2 + lax.axis_index("c")
        pltpu.sync_copy(w0_hbm.at[wid], w0_v)
        pltpu.sync_copy(w1_hbm.at[wid], w1_v)
        for half in range(2):
            pltpu.sync_copy(g0_hbm.at[wid, pl.ds(half * HALF, HALF)], idx_v)
            pltpu.async_copy(ob_hbm.at[idx_v], r0_v, sem).wait()
            pltpu.sync_copy(g1_hbm.at[wid, pl.ds(half * HALF, HALF)], idx_v)
            pltpu.async_copy(ob_hbm.at[idx_v], r1_v, sem).wait()

            def row_body(i, _):
                wv0 = w0_v[half * HALF + i, :]
                wv1 = w1_v[half * HALF + i, :]
                k0 = wv0 != 0.0
                k1 = wv1 != 0.0

                def col_body(c, _):
                    a = r0_v[i, pl.ds(c * 16, 16)]
                    b = r1_v[i, pl.ds(c * 16, 16)]
                    res = (jnp.where(k0, a * wv0, 0.0)
                           + jnp.where(k1, b * wv1, 0.0))
                    r0_v[i, pl.ds(c * 16, 16)] = res
                    return 0

                lax.fori_loop(0, D_MODEL // 16, col_body, 0, unroll=4)
                return 0

            lax.fori_loop(0, HALF, row_body, 0)
            pltpu.sync_copy(r0_v, y_hbm.at[wid, pl.ds(half * HALF, HALF)])

    return combine


def kernel(x, Wr, W1, W2):
    B, T, H = x.shape
    flat_x = x.reshape(N_TOK, H)
    wr_pad = jnp.zeros((EPAD, H), jnp.float32).at[:N_EXPERTS].set(Wr)
    gd0, gd1, gc0, gc1, w0, w1, loss = _run_router(flat_x, wr_pad)

    x32 = flat_x.reshape(NW, TPW, H)
    gd0r = gd0.reshape(NW, TPW)
    gd1r = gd1.reshape(NW, TPW)
    buf = _make_dispatch()(x32, gd0r, gd1r)

    out_buf = _run_ffn(buf, W1, W1, W2)

    gc0r = gc0.reshape(NW, TPW)
    gc1r = gc1.reshape(NW, TPW)
    w0s = jnp.broadcast_to(w0.reshape(N_TOK, 1), (N_TOK, 16)).reshape(NW, TPW, 16)
    w1s = jnp.broadcast_to(w1.reshape(N_TOK, 1), (N_TOK, 16)).reshape(NW, TPW, 16)
    y = _make_combine()(out_buf, gc0r, gc1r, w0s, w1s)

    return y.reshape(B, T, H), loss.reshape(())
